# Initial kernel scaffold; baseline (speedup 1.0000x reference)
import jax
import jax.numpy as jnp
from jax.experimental import pallas as pl


def kernel(qkv, sim, x_size, proj_w, proj_b, logit_scale):
    raise NotImplementedError("write your pallas kernel here")


# speedup vs baseline: 1.6836x; 1.6836x over previous
import functools

import jax
import jax.numpy as jnp
from jax.experimental import pallas as pl
from jax.experimental.pallas import tpu as pltpu

DIM = 256
NUM_HEADS = 8
HEAD_DIM = DIM // NUM_HEADS
GS = 128


def _attn_body(scale_ref, qkv_ref, w_ref, b_ref, out_ref):
    blk = qkv_ref[0]
    scale = jnp.exp(jnp.minimum(scale_ref[0, 0], jnp.log(1.0 / 0.01)))
    q = blk[:, :DIM]
    k = blk[:, DIM:2 * DIM]
    v = blk[:, 2 * DIM:]
    outs = []
    for h in range(NUM_HEADS):
        sl = slice(h * HEAD_DIM, (h + 1) * HEAD_DIM)
        qh, kh, vh = q[:, sl], k[:, sl], v[:, sl]
        s = jax.lax.dot_general(qh, kh, (((1,), (1,)), ((), ())),
                                preferred_element_type=jnp.float32) * scale
        s = s - jnp.max(s, axis=-1, keepdims=True)
        e = jnp.exp(s)
        p = e / jnp.sum(e, axis=-1, keepdims=True)
        outs.append(jax.lax.dot_general(p, vh, (((1,), (0,)), ((), ())),
                                        preferred_element_type=jnp.float32))
    o = jnp.concatenate(outs, axis=1)
    res = jax.lax.dot_general(o, w_ref[...], (((1,), (1,)), ((), ())),
                              preferred_element_type=jnp.float32)
    out_ref[0] = res + b_ref[...]


def _grouped_attention(grouped, proj_w, proj_b, logit_scale):
    nblk = grouped.shape[0]
    return pl.pallas_call(
        _attn_body,
        grid=(nblk,),
        in_specs=[
            pl.BlockSpec(memory_space=pltpu.SMEM),
            pl.BlockSpec((1, GS, 3 * DIM), lambda i: (i, 0, 0)),
            pl.BlockSpec((DIM, DIM), lambda i: (0, 0)),
            pl.BlockSpec((1, DIM), lambda i: (0, 0)),
        ],
        out_specs=pl.BlockSpec((1, GS, DIM), lambda i: (i, 0, 0)),
        out_shape=jax.ShapeDtypeStruct((nblk, GS, DIM), jnp.float32),
    )(logit_scale, grouped, proj_w, proj_b.reshape(1, DIM))


def kernel(qkv, sim, x_size, proj_w, proj_b, logit_scale):
    b, n, c3 = qkv.shape
    ng = n // GS
    tk_id = jnp.argmax(sim, axis=-1)
    sort_idx = jnp.argsort(tk_id, axis=-1, stable=True)
    shuffled = jnp.take_along_axis(qkv, sort_idx[:, :, None], axis=1)
    grouped = shuffled.reshape(b * ng, GS, c3)
    out = _grouped_attention(grouped, proj_w, proj_b, logit_scale)
    out = out.reshape(b, n, DIM)
    inv = jnp.argsort(sort_idx, axis=-1)
    x = jnp.take_along_axis(out, inv[:, :, None], axis=1)
    return x


# speedup vs baseline: 2.4637x; 1.4634x over previous
import functools

import jax
import jax.numpy as jnp
from jax.experimental import pallas as pl
from jax.experimental.pallas import tpu as pltpu

DIM = 256
NUM_HEADS = 8
HEAD_DIM = DIM // NUM_HEADS
GS = 128
GB = 8
LOG2E = 1.4426950408889634


def _attn_body(scale_ref, qkv_ref, w_ref, b_ref, out_ref):
    scale = jnp.exp(jnp.minimum(scale_ref[0, 0], jnp.log(1.0 / 0.01)))
    sl2 = scale * LOG2E
    w = w_ref[...]
    bias = b_ref[...]
    for g in range(GB):
        blk = qkv_ref[g]
        q = blk[:, :DIM] * sl2
        k = blk[:, DIM:2 * DIM]
        v = blk[:, 2 * DIM:]
        outs = []
        for h in range(NUM_HEADS):
            sl = slice(h * HEAD_DIM, (h + 1) * HEAD_DIM)
            s = jax.lax.dot_general(q[:, sl], k[:, sl], (((1,), (1,)), ((), ())),
                                    preferred_element_type=jnp.float32)
            m = jnp.max(s, axis=-1, keepdims=True)
            e = jnp.exp2(s - m)
            r = 1.0 / jnp.sum(e, axis=-1, keepdims=True)
            acc = jax.lax.dot_general(e, v[:, sl], (((1,), (0,)), ((), ())),
                                      preferred_element_type=jnp.float32)
            outs.append(acc * r)
        o = jnp.concatenate(outs, axis=1)
        res = jax.lax.dot_general(o, w, (((1,), (1,)), ((), ())),
                                  preferred_element_type=jnp.float32)
        out_ref[g] = res + bias


def _grouped_attention(grouped, proj_w, proj_b, logit_scale):
    nblk = grouped.shape[0]
    return pl.pallas_call(
        _attn_body,
        grid=(nblk // GB,),
        in_specs=[
            pl.BlockSpec(memory_space=pltpu.SMEM),
            pl.BlockSpec((GB, GS, 3 * DIM), lambda i: (i, 0, 0)),
            pl.BlockSpec((DIM, DIM), lambda i: (0, 0)),
            pl.BlockSpec((1, DIM), lambda i: (0, 0)),
        ],
        out_specs=pl.BlockSpec((GB, GS, DIM), lambda i: (i, 0, 0)),
        out_shape=jax.ShapeDtypeStruct((nblk, GS, DIM), jnp.float32),
        compiler_params=pltpu.CompilerParams(
            dimension_semantics=("arbitrary",)),
    )(logit_scale, grouped, proj_w, proj_b.reshape(1, DIM))


def kernel(qkv, sim, x_size, proj_w, proj_b, logit_scale):
    b, n, c3 = qkv.shape
    ng = n // GS
    tk_id = jnp.argmax(sim, axis=-1)
    sort_idx = jnp.argsort(tk_id, axis=-1, stable=True)
    shuffled = jnp.take_along_axis(qkv, sort_idx[:, :, None], axis=1)
    grouped = shuffled.reshape(b * ng, GS, c3)
    out = _grouped_attention(grouped, proj_w, proj_b, logit_scale)
    out = out.reshape(b, n, DIM)
    inv = jnp.argsort(sort_idx, axis=-1)
    x = jnp.take_along_axis(out, inv[:, :, None], axis=1)
    return x


# speedup vs baseline: 2.4788x; 1.0061x over previous
import functools

import jax
import jax.numpy as jnp
from jax.experimental import pallas as pl
from jax.experimental.pallas import tpu as pltpu

DIM = 256
NUM_HEADS = 8
HEAD_DIM = DIM // NUM_HEADS
GS = 128
GB = 8
LOG2E = 1.4426950408889634


def _attn_body(scale_ref, qkv_ref, w_ref, b_ref, out_ref):
    scale = jnp.exp(jnp.minimum(scale_ref[0, 0], jnp.log(1.0 / 0.01)))
    sl2 = scale * LOG2E
    w = w_ref[...]
    bias = b_ref[...]
    for g in range(GB):
        blk = qkv_ref[g]
        q = blk[:, :DIM]
        k = blk[:, DIM:2 * DIM]
        v = blk[:, 2 * DIM:]
        outs = []
        for h in range(NUM_HEADS):
            sl = slice(h * HEAD_DIM, (h + 1) * HEAD_DIM)
            s = jax.lax.dot_general(q[:, sl], k[:, sl], (((1,), (1,)), ((), ())),
                                    preferred_element_type=jnp.float32) * sl2
            m = jnp.max(s, axis=-1, keepdims=True)
            e = jnp.exp2(s - m)
            r = 1.0 / jnp.sum(e, axis=-1, keepdims=True)
            acc = jax.lax.dot_general(e, v[:, sl], (((1,), (0,)), ((), ())),
                                      preferred_element_type=jnp.float32)
            outs.append(acc * r)
        o = jnp.concatenate(outs, axis=1)
        res = jax.lax.dot_general(o, w, (((1,), (1,)), ((), ())),
                                  preferred_element_type=jnp.float32)
        out_ref[g] = res + bias


def _grouped_attention(grouped, proj_w, proj_b, logit_scale):
    nblk = grouped.shape[0]
    return pl.pallas_call(
        _attn_body,
        grid=(nblk // GB,),
        in_specs=[
            pl.BlockSpec(memory_space=pltpu.SMEM),
            pl.BlockSpec((GB, GS, 3 * DIM), lambda i: (i, 0, 0)),
            pl.BlockSpec((DIM, DIM), lambda i: (0, 0)),
            pl.BlockSpec((1, DIM), lambda i: (0, 0)),
        ],
        out_specs=pl.BlockSpec((GB, GS, DIM), lambda i: (i, 0, 0)),
        out_shape=jax.ShapeDtypeStruct((nblk, GS, DIM), jnp.float32),
        compiler_params=pltpu.CompilerParams(
            dimension_semantics=("arbitrary",)),
    )(logit_scale, grouped, proj_w, proj_b.reshape(1, DIM))


def kernel(qkv, sim, x_size, proj_w, proj_b, logit_scale):
    b, n, c3 = qkv.shape
    ng = n // GS
    tk_id = jnp.argmax(sim, axis=-1)
    sort_idx = jnp.argsort(tk_id, axis=-1, stable=True)
    shuffled = jnp.take_along_axis(qkv, sort_idx[:, :, None], axis=1)
    grouped = shuffled.reshape(b * ng, GS, c3)
    out = _grouped_attention(grouped, proj_w, proj_b, logit_scale)
    out = out.reshape(b, n, DIM)
    inv = jnp.argsort(sort_idx, axis=-1)
    x = jnp.take_along_axis(out, inv[:, :, None], axis=1)
    return x
